# baseline (speedup 1.0000x reference)
import functools

import jax
import jax.numpy as jnp
from jax import lax
from jax.experimental import pallas as pl
from jax.experimental.pallas import tpu as pltpu
from jax.experimental.pallas import tpu_sc as plsc

VOCAB_P1 = 1001
DIM = 128
BATCH = 16384

_NC = 2
_NS = 16
_NW = _NC * _NS
_BPW = BATCH // _NW
_CHUNK = 128
_NCH = _BPW // _CHUNK
_RPT = 64

_mesh = plsc.VectorSubcoreMesh(core_axis_name="c", subcore_axis_name="s")


@functools.partial(
    pl.kernel,
    mesh=_mesh,
    out_type=jax.ShapeDtypeStruct((BATCH, DIM), jnp.float32),
    scratch_types=[
        pltpu.VMEM((_NCH, _CHUNK), jnp.int32),
        pltpu.VMEM((_BPW, DIM), jnp.float32),
        pltpu.VMEM_SHARED((1008, DIM), jnp.float32),
        pltpu.SemaphoreType.DMA,
        pltpu.SemaphoreType.DMA,
        pltpu.SemaphoreType.DMA,
    ],
)
def _gather_kernel(table_hbm, idx_hbm, out_hbm, idx_v, rows_v, table_sh,
                   ssem, gsem, osem):
    sid = lax.axis_index("s")
    wid = sid * _NC + lax.axis_index("c")
    off = jnp.minimum(sid * _RPT, 936)
    stage = pltpu.async_copy(
        table_hbm.at[pl.ds(off, _RPT)], table_sh.at[pl.ds(off, _RPT)], ssem
    )

    @pl.when(sid == 0)
    def _():
        pltpu.sync_copy(table_hbm.at[pl.ds(1000, 1)],
                        table_sh.at[pl.ds(1000, 1)])

    pltpu.sync_copy(idx_hbm.at[pl.ds(wid * _NCH, _NCH)], idx_v)
    stage.wait()
    plsc.subcore_barrier()
    gathers = []
    for j in range(_NCH):
        gathers.append(
            pltpu.async_copy(
                table_sh.at[idx_v.at[j]],
                rows_v.at[pl.ds(j * _CHUNK, _CHUNK)],
                gsem,
            )
        )
    writes = []
    for j in range(0, _NCH, 2):
        gathers[j].wait()
        gathers[j + 1].wait()
        writes.append(
            pltpu.async_copy(
                rows_v.at[pl.ds(j * _CHUNK, 2 * _CHUNK)],
                out_hbm.at[pl.ds(wid * _BPW + j * _CHUNK, 2 * _CHUNK)],
                osem,
            )
        )
    for w in writes:
        w.wait()


def kernel(table, call_idx):
    idx2d = call_idx.reshape(BATCH // _CHUNK, _CHUNK)
    return _gather_kernel(table, idx2d)


# speedup vs baseline: 1.0030x; 1.0030x over previous
import functools

import jax
import jax.numpy as jnp
from jax import lax
from jax.experimental import pallas as pl
from jax.experimental.pallas import tpu as pltpu
from jax.experimental.pallas import tpu_sc as plsc

VOCAB_P1 = 1001
DIM = 128
BATCH = 16384

_NC = 2
_NS = 16
_NW = _NC * _NS
_BPW = BATCH // _NW
_CHUNK = 64
_NCH = _BPW // _CHUNK
_RPT = 64

_mesh = plsc.VectorSubcoreMesh(core_axis_name="c", subcore_axis_name="s")


@functools.partial(
    pl.kernel,
    mesh=_mesh,
    out_type=jax.ShapeDtypeStruct((BATCH, DIM), jnp.float32),
    scratch_types=[
        pltpu.VMEM((_NCH, _CHUNK), jnp.int32),
        pltpu.VMEM((_BPW, DIM), jnp.float32),
        pltpu.VMEM_SHARED((1008, DIM), jnp.float32),
        pltpu.SemaphoreType.DMA,
        pltpu.SemaphoreType.DMA,
        pltpu.SemaphoreType.DMA,
    ],
)
def _gather_kernel(table_hbm, idx_hbm, out_hbm, idx_v, rows_v, table_sh,
                   ssem, gsem, osem):
    sid = lax.axis_index("s")
    wid = sid * _NC + lax.axis_index("c")
    off = jnp.minimum(sid * _RPT, 936)
    stage = pltpu.async_copy(
        table_hbm.at[pl.ds(off, _RPT)], table_sh.at[pl.ds(off, _RPT)], ssem
    )

    @pl.when(sid == 0)
    def _():
        pltpu.sync_copy(table_hbm.at[pl.ds(1000, 1)],
                        table_sh.at[pl.ds(1000, 1)])

    pltpu.sync_copy(idx_hbm.at[pl.ds(wid * _NCH, _NCH)], idx_v)
    stage.wait()
    plsc.subcore_barrier()
    gathers = []
    for j in range(_NCH):
        gathers.append(
            pltpu.async_copy(
                table_sh.at[idx_v.at[j]],
                rows_v.at[pl.ds(j * _CHUNK, _CHUNK)],
                gsem,
            )
        )
    writes = []
    for j in range(_NCH):
        gathers[j].wait()
        writes.append(
            pltpu.async_copy(
                rows_v.at[pl.ds(j * _CHUNK, _CHUNK)],
                out_hbm.at[pl.ds(wid * _BPW + j * _CHUNK, _CHUNK)],
                osem,
            )
        )
    for w in writes:
        w.wait()


def kernel(table, call_idx):
    idx2d = call_idx.reshape(BATCH // _CHUNK, _CHUNK)
    return _gather_kernel(table, idx2d)


# speedup vs baseline: 1.0173x; 1.0143x over previous
import functools

import jax
import jax.numpy as jnp
from jax import lax
from jax.experimental import pallas as pl
from jax.experimental.pallas import tpu as pltpu
from jax.experimental.pallas import tpu_sc as plsc

VOCAB_P1 = 1001
DIM = 128
BATCH = 16384

_NC = 2
_NS = 16
_NW = _NC * _NS
_BPW = BATCH // _NW
_CHUNK = 128
_NCH = _BPW // _CHUNK
_RPT = 64

_mesh = plsc.VectorSubcoreMesh(core_axis_name="c", subcore_axis_name="s")


@functools.partial(
    pl.kernel,
    mesh=_mesh,
    out_type=jax.ShapeDtypeStruct((BATCH, DIM), jnp.float32),
    compiler_params=pltpu.CompilerParams(skip_device_barrier=True),
    scratch_types=[
        pltpu.VMEM((_NCH, _CHUNK), jnp.int32),
        pltpu.VMEM((_BPW, DIM), jnp.float32),
        pltpu.VMEM_SHARED((1008, DIM), jnp.float32),
        pltpu.SemaphoreType.DMA,
        pltpu.SemaphoreType.DMA,
        pltpu.SemaphoreType.DMA,
    ],
)
def _gather_kernel(table_hbm, idx_hbm, out_hbm, idx_v, rows_v, table_sh,
                   ssem, gsem, osem):
    sid = lax.axis_index("s")
    wid = sid * _NC + lax.axis_index("c")
    off = jnp.minimum(sid * _RPT, 936)
    stage = pltpu.async_copy(
        table_hbm.at[pl.ds(off, _RPT)], table_sh.at[pl.ds(off, _RPT)], ssem
    )

    @pl.when(sid == 0)
    def _():
        pltpu.sync_copy(table_hbm.at[pl.ds(1000, 1)],
                        table_sh.at[pl.ds(1000, 1)])

    pltpu.sync_copy(idx_hbm.at[pl.ds(wid * _NCH, _NCH)], idx_v)
    stage.wait()
    plsc.subcore_barrier()
    gathers = []
    for j in range(_NCH):
        gathers.append(
            pltpu.async_copy(
                table_sh.at[idx_v.at[j]],
                rows_v.at[pl.ds(j * _CHUNK, _CHUNK)],
                gsem,
            )
        )
    writes = []
    for j in range(_NCH):
        gathers[j].wait()
        writes.append(
            pltpu.async_copy(
                rows_v.at[pl.ds(j * _CHUNK, _CHUNK)],
                out_hbm.at[pl.ds(wid * _BPW + j * _CHUNK, _CHUNK)],
                osem,
            )
        )
    for w in writes:
        w.wait()


def kernel(table, call_idx):
    idx2d = call_idx.reshape(BATCH // _CHUNK, _CHUNK)
    return _gather_kernel(table, idx2d)
